# baseline (speedup 1.0000x reference)
import functools

import jax
import jax.numpy as jnp
from jax import lax
from jax.experimental import pallas as pl
from jax.experimental.pallas import tpu as pltpu
from jax.experimental.pallas import tpu_sc as plsc

_SC_ROWS = 2048
_TC_BLK = 512


def _shuf(v, idx):
    return lax.gather(
        v,
        idx[:, None],
        dimension_numbers=lax.GatherDimensionNumbers(
            offset_dims=(), collapsed_slice_dims=(0,), start_index_map=(0,)
        ),
        slice_sizes=(1,),
        mode=lax.GatherScatterMode.PROMISE_IN_BOUNDS,
    )


def _argmax_body(x_ref, tok_ref):
    xb = x_ref[...]
    m = jnp.max(xb, axis=-1, keepdims=True)
    ii = lax.broadcasted_iota(jnp.int32, xb.shape, 1)
    tok_ref[...] = jnp.min(jnp.where(xb == m, ii, xb.shape[-1]), axis=-1)


@functools.lru_cache(maxsize=None)
def _make_sc_argmax(S, V, row0):
    info = plsc.get_sparse_core_info()
    NC, NS, L = info.num_cores, info.num_subcores, info.num_lanes
    NW = NC * NS
    rpw = S // NW
    nchunks = V // L
    assert S % NW == 0 and rpw % 2 == 0 and rpw % 8 == 0
    mesh = plsc.VectorSubcoreMesh(core_axis_name="c", subcore_axis_name="s")

    @functools.partial(
        pl.kernel,
        mesh=mesh,
        out_type=jax.ShapeDtypeStruct((S,), jnp.int32),
        scratch_types=[
            pltpu.VMEM((2, V), jnp.float32),
            pltpu.VMEM((rpw,), jnp.int32),
            pltpu.SemaphoreType.DMA,
            pltpu.SemaphoreType.DMA,
        ],
    )
    def sc_argmax(x_hbm, out_hbm, buf, tok_v, sem0, sem1):
        wid = lax.axis_index("s") * NC + lax.axis_index("c")
        obase = wid * rpw
        base = row0 + obase
        lane = lax.iota(jnp.int32, L)
        neg_inf = jnp.full((L,), -jnp.inf, jnp.float32)
        zero_i = jnp.zeros((L,), jnp.int32)

        def row_argmax(slot):
            row = buf.at[slot]
            K = 8

            def block(j, carry):
                ms, js = carry
                ms, js = list(ms), list(js)
                for k in range(K):
                    jg = j * K + k
                    v = row[pl.ds(jg * L, L)]
                    c = v > ms[k]
                    ms[k] = jnp.where(c, v, ms[k])
                    js[k] = jnp.where(c, jg, js[k])
                return tuple(ms), tuple(js)

            ms, js = lax.fori_loop(
                0,
                nchunks // K,
                block,
                ((neg_inf,) * K, (zero_i,) * K),
                unroll=4,
            )
            ms, js = list(ms), list(js)
            step = 1
            while step < K:
                for a in range(0, K, 2 * step):
                    b = a + step
                    c = (ms[b] > ms[a]) | ((ms[b] == ms[a]) & (js[b] < js[a]))
                    ms[a] = jnp.where(c, ms[b], ms[a])
                    js[a] = jnp.where(c, js[b], js[a])
                step *= 2
            m_run, i_run = ms[0], js[0]
            m = m_run
            for k in (8, 4, 2, 1):
                m = jnp.maximum(m, _shuf(m, lane ^ k))
            cand = jnp.where(m_run == m, i_run * L + lane, jnp.int32(V))
            for k in (8, 4, 2, 1):
                cand = jnp.minimum(cand, _shuf(cand, lane ^ k))
            return cand

        pltpu.async_copy(x_hbm.at[base], buf.at[0], sem0)
        pltpu.async_copy(x_hbm.at[base + 1], buf.at[1], sem1)

        def pair(p, acc):
            r0 = 2 * p
            pltpu.make_async_copy(x_hbm.at[base + r0], buf.at[0], sem0).wait()
            acc = jnp.where(lane == r0 % L, row_argmax(0), acc)

            @pl.when(p + 1 < rpw // 2)
            def _():
                pltpu.async_copy(x_hbm.at[base + r0 + 2], buf.at[0], sem0)

            pltpu.make_async_copy(x_hbm.at[base + r0 + 1], buf.at[1], sem1).wait()
            acc = jnp.where(lane == (r0 + 1) % L, row_argmax(1), acc)

            @pl.when(p + 1 < rpw // 2)
            def _():
                pltpu.async_copy(x_hbm.at[base + r0 + 3], buf.at[1], sem1)

            @pl.when((r0 + 1) % L == L - 1)
            def _():
                tok_v[pl.ds(r0 + 1 - (L - 1), L)] = acc

            return acc

        lax.fori_loop(0, rpw // 2, pair, zero_i)
        pltpu.sync_copy(tok_v, out_hbm.at[pl.ds(obase, rpw)])

    return sc_argmax


@functools.lru_cache(maxsize=None)
def _make_sc_gather(V, D, B):
    info = plsc.get_sparse_core_info()
    NC, NS = info.num_cores, info.num_subcores
    NW = NC * NS
    b_per_w = B // NW
    assert B % (8 * NW) == 0 and D % info.num_lanes == 0
    mesh = plsc.VectorSubcoreMesh(core_axis_name="c", subcore_axis_name="s")

    @functools.partial(
        pl.kernel,
        mesh=mesh,
        out_type=jax.ShapeDtypeStruct((B, D), jnp.float32),
        scratch_types=[
            pltpu.VMEM((b_per_w,), jnp.int32),
            pltpu.VMEM((b_per_w, D), jnp.float32),
            pltpu.SemaphoreType.DMA,
        ],
    )
    def gather(table_hbm, idx_hbm, out_hbm, idx_v, rows_v, sem):
        wid = lax.axis_index("s") * NC + lax.axis_index("c")
        base = wid * b_per_w
        pltpu.sync_copy(idx_hbm.at[pl.ds(base, b_per_w)], idx_v)
        pltpu.async_copy(table_hbm.at[idx_v], rows_v, sem).wait()
        pltpu.sync_copy(rows_v, out_hbm.at[pl.ds(base, b_per_w)])

    return gather


@jax.jit
def kernel(x, dictionary):
    B, N, V = x.shape
    D = dictionary.shape[1]
    R = B * N
    xf = x.reshape(R, V)

    S = _SC_ROWS
    R_tc = R - S
    tokens_tc = pl.pallas_call(
        _argmax_body,
        grid=(R_tc // _TC_BLK,),
        in_specs=[pl.BlockSpec((_TC_BLK, V), lambda i: (i, 0))],
        out_specs=pl.BlockSpec((_TC_BLK,), lambda i: (i,)),
        out_shape=jax.ShapeDtypeStruct((R_tc,), jnp.int32),
    )(xf)
    tokens_sc = _make_sc_argmax(S, V, R_tc)(xf)
    tokens = jnp.concatenate([tokens_tc, tokens_sc])

    DP = 128
    table = jnp.pad(dictionary, ((0, 0), (0, DP - D)))
    out = _make_sc_gather(V, DP, R)(table, tokens)
    return out[:, :D].reshape(B, N, D)


# speedup vs baseline: 1.0067x; 1.0067x over previous
import functools

import jax
import jax.numpy as jnp
from jax import lax
from jax.experimental import pallas as pl
from jax.experimental.pallas import tpu as pltpu
from jax.experimental.pallas import tpu_sc as plsc

_SC_ROWS = 2560
_TC_BLK = 512


def _shuf(v, idx):
    return lax.gather(
        v,
        idx[:, None],
        dimension_numbers=lax.GatherDimensionNumbers(
            offset_dims=(), collapsed_slice_dims=(0,), start_index_map=(0,)
        ),
        slice_sizes=(1,),
        mode=lax.GatherScatterMode.PROMISE_IN_BOUNDS,
    )


def _argmax_body(x_ref, tok_ref):
    xb = x_ref[...]
    m = jnp.max(xb, axis=-1, keepdims=True)
    ii = lax.broadcasted_iota(jnp.int32, xb.shape, 1)
    tok_ref[...] = jnp.min(jnp.where(xb == m, ii, xb.shape[-1]), axis=-1)


@functools.lru_cache(maxsize=None)
def _make_sc_argmax(S, V, row0):
    info = plsc.get_sparse_core_info()
    NC, NS, L = info.num_cores, info.num_subcores, info.num_lanes
    NW = NC * NS
    rpw = S // NW
    nchunks = V // L
    assert S % NW == 0 and rpw % 2 == 0 and rpw % 8 == 0
    mesh = plsc.VectorSubcoreMesh(core_axis_name="c", subcore_axis_name="s")

    @functools.partial(
        pl.kernel,
        mesh=mesh,
        out_type=jax.ShapeDtypeStruct((S,), jnp.int32),
        scratch_types=[
            pltpu.VMEM((2, V), jnp.float32),
            pltpu.VMEM((rpw,), jnp.int32),
            pltpu.SemaphoreType.DMA,
            pltpu.SemaphoreType.DMA,
        ],
    )
    def sc_argmax(x_hbm, out_hbm, buf, tok_v, sem0, sem1):
        wid = lax.axis_index("s") * NC + lax.axis_index("c")
        obase = wid * rpw
        base = row0 + obase
        lane = lax.iota(jnp.int32, L)
        neg_inf = jnp.full((L,), -jnp.inf, jnp.float32)
        zero_i = jnp.zeros((L,), jnp.int32)

        def row_argmax(slot):
            row = buf.at[slot]
            K = 8

            def block(j, carry):
                ms, js = carry
                ms, js = list(ms), list(js)
                for k in range(K):
                    jg = j * K + k
                    v = row[pl.ds(jg * L, L)]
                    c = v > ms[k]
                    ms[k] = jnp.where(c, v, ms[k])
                    js[k] = jnp.where(c, jg, js[k])
                return tuple(ms), tuple(js)

            ms, js = lax.fori_loop(
                0,
                nchunks // K,
                block,
                ((neg_inf,) * K, (zero_i,) * K),
            )
            ms, js = list(ms), list(js)
            step = 1
            while step < K:
                for a in range(0, K, 2 * step):
                    b = a + step
                    c = (ms[b] > ms[a]) | ((ms[b] == ms[a]) & (js[b] < js[a]))
                    ms[a] = jnp.where(c, ms[b], ms[a])
                    js[a] = jnp.where(c, js[b], js[a])
                step *= 2
            m_run, i_run = ms[0], js[0]
            m = m_run
            for k in (8, 4, 2, 1):
                m = jnp.maximum(m, _shuf(m, lane ^ k))
            cand = jnp.where(m_run == m, i_run * L + lane, jnp.int32(V))
            for k in (8, 4, 2, 1):
                cand = jnp.minimum(cand, _shuf(cand, lane ^ k))
            return cand

        pltpu.async_copy(x_hbm.at[base], buf.at[0], sem0)
        pltpu.async_copy(x_hbm.at[base + 1], buf.at[1], sem1)

        def pair(p, acc):
            r0 = 2 * p
            pltpu.make_async_copy(x_hbm.at[base + r0], buf.at[0], sem0).wait()
            acc = jnp.where(lane == r0 % L, row_argmax(0), acc)

            @pl.when(p + 1 < rpw // 2)
            def _():
                pltpu.async_copy(x_hbm.at[base + r0 + 2], buf.at[0], sem0)

            pltpu.make_async_copy(x_hbm.at[base + r0 + 1], buf.at[1], sem1).wait()
            acc = jnp.where(lane == (r0 + 1) % L, row_argmax(1), acc)

            @pl.when(p + 1 < rpw // 2)
            def _():
                pltpu.async_copy(x_hbm.at[base + r0 + 3], buf.at[1], sem1)

            @pl.when((r0 + 1) % L == L - 1)
            def _():
                tok_v[pl.ds(r0 + 1 - (L - 1), L)] = acc

            return acc

        lax.fori_loop(0, rpw // 2, pair, zero_i)
        pltpu.sync_copy(tok_v, out_hbm.at[pl.ds(obase, rpw)])

    return sc_argmax


@functools.lru_cache(maxsize=None)
def _make_sc_gather(V, D, B):
    info = plsc.get_sparse_core_info()
    NC, NS = info.num_cores, info.num_subcores
    NW = NC * NS
    b_per_w = B // NW
    assert B % (8 * NW) == 0 and D % info.num_lanes == 0
    mesh = plsc.VectorSubcoreMesh(core_axis_name="c", subcore_axis_name="s")

    @functools.partial(
        pl.kernel,
        mesh=mesh,
        out_type=jax.ShapeDtypeStruct((B, D), jnp.float32),
        scratch_types=[
            pltpu.VMEM((b_per_w,), jnp.int32),
            pltpu.VMEM((b_per_w, D), jnp.float32),
            pltpu.SemaphoreType.DMA,
        ],
    )
    def gather(table_hbm, idx_hbm, out_hbm, idx_v, rows_v, sem):
        wid = lax.axis_index("s") * NC + lax.axis_index("c")
        base = wid * b_per_w
        pltpu.sync_copy(idx_hbm.at[pl.ds(base, b_per_w)], idx_v)
        pltpu.async_copy(table_hbm.at[idx_v], rows_v, sem).wait()
        pltpu.sync_copy(rows_v, out_hbm.at[pl.ds(base, b_per_w)])

    return gather


@jax.jit
def kernel(x, dictionary):
    B, N, V = x.shape
    D = dictionary.shape[1]
    R = B * N
    xf = x.reshape(R, V)

    S = _SC_ROWS
    R_tc = R - S
    tokens_tc = pl.pallas_call(
        _argmax_body,
        grid=(R_tc // _TC_BLK,),
        in_specs=[pl.BlockSpec((_TC_BLK, V), lambda i: (i, 0))],
        out_specs=pl.BlockSpec((_TC_BLK,), lambda i: (i,)),
        out_shape=jax.ShapeDtypeStruct((R_tc,), jnp.int32),
    )(xf)
    tokens_sc = _make_sc_argmax(S, V, R_tc)(xf)
    tokens = jnp.concatenate([tokens_tc, tokens_sc])

    DP = 128
    table = jnp.pad(dictionary, ((0, 0), (0, DP - D)))
    out = _make_sc_gather(V, DP, R)(table, tokens)
    return out[:, :D].reshape(B, N, D)


# speedup vs baseline: 1.0217x; 1.0149x over previous
import functools

import jax
import jax.numpy as jnp
from jax import lax
from jax.experimental import pallas as pl
from jax.experimental.pallas import tpu as pltpu
from jax.experimental.pallas import tpu_sc as plsc

_SC_ROWS = 2560
_TC_BLK = 512


def _shuf(v, idx):
    return lax.gather(
        v,
        idx[:, None],
        dimension_numbers=lax.GatherDimensionNumbers(
            offset_dims=(), collapsed_slice_dims=(0,), start_index_map=(0,)
        ),
        slice_sizes=(1,),
        mode=lax.GatherScatterMode.PROMISE_IN_BOUNDS,
    )


def _argmax_body(x_ref, tok_ref):
    xb = x_ref[...]
    m = jnp.max(xb, axis=-1, keepdims=True)
    ii = lax.broadcasted_iota(jnp.int32, xb.shape, 1)
    tok_ref[...] = jnp.min(jnp.where(xb == m, ii, xb.shape[-1]), axis=-1)


@functools.lru_cache(maxsize=None)
def _make_sc_argmax(S, V, row0):
    info = plsc.get_sparse_core_info()
    NC, NS, L = info.num_cores, info.num_subcores, info.num_lanes
    NW = NC * NS
    rpw = S // NW
    nchunks = V // L
    assert S % NW == 0 and rpw % 2 == 0 and rpw % 8 == 0
    mesh = plsc.VectorSubcoreMesh(core_axis_name="c", subcore_axis_name="s")

    @functools.partial(
        pl.kernel,
        mesh=mesh,
        out_type=jax.ShapeDtypeStruct((S,), jnp.int32),
        scratch_types=[
            pltpu.VMEM((2, V), jnp.float32),
            pltpu.VMEM((rpw,), jnp.int32),
            pltpu.SemaphoreType.DMA,
            pltpu.SemaphoreType.DMA,
        ],
    )
    def sc_argmax(x_hbm, out_hbm, buf, tok_v, sem0, sem1):
        wid = lax.axis_index("s") * NC + lax.axis_index("c")
        obase = wid * rpw
        base = row0 + obase
        lane = lax.iota(jnp.int32, L)
        neg_inf = jnp.full((L,), -jnp.inf, jnp.float32)
        zero_i = jnp.zeros((L,), jnp.int32)

        def row_argmax(slot):
            row = buf.at[slot]
            K = 8

            def block(j, carry):
                ms, js = carry
                ms, js = list(ms), list(js)
                for k in range(K):
                    jg = j * K + k
                    v = row[pl.ds(jg * L, L)]
                    c = v > ms[k]
                    ms[k] = jnp.where(c, v, ms[k])
                    js[k] = jnp.where(c, jg, js[k])
                return tuple(ms), tuple(js)

            ms, js = lax.fori_loop(
                0,
                nchunks // K,
                block,
                ((neg_inf,) * K, (zero_i,) * K),
            )
            ms, js = list(ms), list(js)
            step = 1
            while step < K:
                for a in range(0, K, 2 * step):
                    b = a + step
                    c = (ms[b] > ms[a]) | ((ms[b] == ms[a]) & (js[b] < js[a]))
                    ms[a] = jnp.where(c, ms[b], ms[a])
                    js[a] = jnp.where(c, js[b], js[a])
                step *= 2
            m_run, i_run = ms[0], js[0]
            m = m_run
            for k in (8, 4, 2, 1):
                m = jnp.maximum(m, _shuf(m, lane ^ k))
            cand = jnp.where(m_run == m, i_run * L + lane, jnp.int32(V))
            for k in (8, 4, 2, 1):
                cand = jnp.minimum(cand, _shuf(cand, lane ^ k))
            return cand

        pltpu.async_copy(x_hbm.at[base], buf.at[0], sem0)
        pltpu.async_copy(x_hbm.at[base + 1], buf.at[1], sem1)

        def pair(p, acc):
            r0 = 2 * p
            pltpu.make_async_copy(x_hbm.at[base + r0], buf.at[0], sem0).wait()
            acc = jnp.where(lane == r0 % L, row_argmax(0), acc)

            @pl.when(p + 1 < rpw // 2)
            def _():
                pltpu.async_copy(x_hbm.at[base + r0 + 2], buf.at[0], sem0)

            pltpu.make_async_copy(x_hbm.at[base + r0 + 1], buf.at[1], sem1).wait()
            acc = jnp.where(lane == (r0 + 1) % L, row_argmax(1), acc)

            @pl.when(p + 1 < rpw // 2)
            def _():
                pltpu.async_copy(x_hbm.at[base + r0 + 3], buf.at[1], sem1)

            @pl.when((r0 + 1) % L == L - 1)
            def _():
                tok_v[pl.ds(r0 + 1 - (L - 1), L)] = acc

            return acc

        lax.fori_loop(0, rpw // 2, pair, zero_i)
        pltpu.sync_copy(tok_v, out_hbm.at[pl.ds(obase, rpw)])

    return sc_argmax


@functools.lru_cache(maxsize=None)
def _make_sc_gather(V, DP, D, B, B_tc):
    info = plsc.get_sparse_core_info()
    NC, NS = info.num_cores, info.num_subcores
    NW = NC * NS
    b_per_w = B // NW
    w_split = B_tc // b_per_w
    assert B % (8 * NW) == 0 and B_tc % b_per_w == 0 and DP % info.num_lanes == 0
    mesh = plsc.VectorSubcoreMesh(core_axis_name="c", subcore_axis_name="s")

    @functools.partial(
        pl.kernel,
        mesh=mesh,
        out_type=jax.ShapeDtypeStruct((B, DP), jnp.float32),
        scratch_types=[
            pltpu.VMEM((b_per_w,), jnp.int32),
            pltpu.VMEM((b_per_w, DP), jnp.float32),
            pltpu.SemaphoreType.DMA,
        ],
    )
    def gather(table_hbm, idx_tc_hbm, idx_sc_hbm, out_hbm, idx_v, rows_v, sem):
        wid = lax.axis_index("s") * NC + lax.axis_index("c")
        base = wid * b_per_w

        @pl.when(wid < w_split)
        def _():
            pltpu.sync_copy(idx_tc_hbm.at[pl.ds(base, b_per_w)], idx_v)

        @pl.when(wid >= w_split)
        def _():
            pltpu.sync_copy(
                idx_sc_hbm.at[pl.ds(base - B_tc, b_per_w)], idx_v
            )

        pltpu.async_copy(table_hbm.at[idx_v], rows_v, sem).wait()
        pltpu.sync_copy(rows_v, out_hbm.at[pl.ds(base, b_per_w)])

    return gather


@jax.jit
def kernel(x, dictionary):
    B, N, V = x.shape
    D = dictionary.shape[1]
    R = B * N
    xf = x.reshape(R, V)

    S = _SC_ROWS
    R_tc = R - S
    tokens_tc = pl.pallas_call(
        _argmax_body,
        grid=(R_tc // _TC_BLK,),
        in_specs=[pl.BlockSpec((_TC_BLK, V), lambda i: (i, 0))],
        out_specs=pl.BlockSpec((_TC_BLK,), lambda i: (i,)),
        out_shape=jax.ShapeDtypeStruct((R_tc,), jnp.int32),
    )(xf)
    tokens_sc = _make_sc_argmax(S, V, R_tc)(xf)

    DP = 128
    table = jnp.pad(dictionary, ((0, 0), (0, DP - D)))
    out = _make_sc_gather(V, DP, D, R, R_tc)(table, tokens_tc, tokens_sc)
    return out[:, :D].reshape(B, N, D)


# speedup vs baseline: 1.0230x; 1.0012x over previous
import functools

import jax
import jax.numpy as jnp
from jax import lax
from jax.experimental import pallas as pl
from jax.experimental.pallas import tpu as pltpu
from jax.experimental.pallas import tpu_sc as plsc

_SC_ROWS = 2560
_TC_BLK = 512


def _shuf(v, idx):
    return lax.gather(
        v,
        idx[:, None],
        dimension_numbers=lax.GatherDimensionNumbers(
            offset_dims=(), collapsed_slice_dims=(0,), start_index_map=(0,)
        ),
        slice_sizes=(1,),
        mode=lax.GatherScatterMode.PROMISE_IN_BOUNDS,
    )


def _argmax_body(x_ref, tok_ref):
    xb = x_ref[...]
    m = jnp.max(xb, axis=-1, keepdims=True)
    ii = lax.broadcasted_iota(jnp.int32, xb.shape, 1)
    tok_ref[...] = jnp.min(jnp.where(xb == m, ii, xb.shape[-1]), axis=-1)


@functools.lru_cache(maxsize=None)
def _make_sc_argmax(S, V, row0):
    info = plsc.get_sparse_core_info()
    NC, NS, L = info.num_cores, info.num_subcores, info.num_lanes
    NW = NC * NS
    rpw = S // NW
    nchunks = V // L
    assert S % NW == 0 and rpw % 2 == 0 and rpw % 8 == 0
    mesh = plsc.VectorSubcoreMesh(core_axis_name="c", subcore_axis_name="s")

    @functools.partial(
        pl.kernel,
        mesh=mesh,
        out_type=jax.ShapeDtypeStruct((S,), jnp.int32),
        scratch_types=[
            pltpu.VMEM((2, V), jnp.float32),
            pltpu.VMEM((rpw,), jnp.int32),
            pltpu.SemaphoreType.DMA,
            pltpu.SemaphoreType.DMA,
        ],
    )
    def sc_argmax(x_hbm, out_hbm, buf, tok_v, sem0, sem1):
        wid = lax.axis_index("s") * NC + lax.axis_index("c")
        obase = wid * rpw
        base = row0 + obase
        lane = lax.iota(jnp.int32, L)
        neg_inf = jnp.full((L,), -jnp.inf, jnp.float32)
        zero_i = jnp.zeros((L,), jnp.int32)

        def row_argmax(slot):
            row = buf.at[slot]
            K = 16

            def block(j, carry):
                ms, js = carry
                ms, js = list(ms), list(js)
                for k in range(K):
                    jg = j * K + k
                    v = row[pl.ds(jg * L, L)]
                    c = v > ms[k]
                    ms[k] = jnp.where(c, v, ms[k])
                    js[k] = jnp.where(c, jg, js[k])
                return tuple(ms), tuple(js)

            ms, js = lax.fori_loop(
                0,
                nchunks // K,
                block,
                ((neg_inf,) * K, (zero_i,) * K),
            )
            ms, js = list(ms), list(js)
            step = 1
            while step < K:
                for a in range(0, K, 2 * step):
                    b = a + step
                    c = (ms[b] > ms[a]) | ((ms[b] == ms[a]) & (js[b] < js[a]))
                    ms[a] = jnp.where(c, ms[b], ms[a])
                    js[a] = jnp.where(c, js[b], js[a])
                step *= 2
            m_run, i_run = ms[0], js[0]
            m = m_run
            for k in (8, 4, 2, 1):
                m = jnp.maximum(m, _shuf(m, lane ^ k))
            cand = jnp.where(m_run == m, i_run * L + lane, jnp.int32(V))
            for k in (8, 4, 2, 1):
                cand = jnp.minimum(cand, _shuf(cand, lane ^ k))
            return cand

        pltpu.async_copy(x_hbm.at[base], buf.at[0], sem0)
        pltpu.async_copy(x_hbm.at[base + 1], buf.at[1], sem1)

        def pair(p, acc):
            r0 = 2 * p
            pltpu.make_async_copy(x_hbm.at[base + r0], buf.at[0], sem0).wait()
            acc = jnp.where(lane == r0 % L, row_argmax(0), acc)

            @pl.when(p + 1 < rpw // 2)
            def _():
                pltpu.async_copy(x_hbm.at[base + r0 + 2], buf.at[0], sem0)

            pltpu.make_async_copy(x_hbm.at[base + r0 + 1], buf.at[1], sem1).wait()
            acc = jnp.where(lane == (r0 + 1) % L, row_argmax(1), acc)

            @pl.when(p + 1 < rpw // 2)
            def _():
                pltpu.async_copy(x_hbm.at[base + r0 + 3], buf.at[1], sem1)

            @pl.when((r0 + 1) % L == L - 1)
            def _():
                tok_v[pl.ds(r0 + 1 - (L - 1), L)] = acc

            return acc

        lax.fori_loop(0, rpw // 2, pair, zero_i)
        pltpu.sync_copy(tok_v, out_hbm.at[pl.ds(obase, rpw)])

    return sc_argmax


@functools.lru_cache(maxsize=None)
def _make_sc_gather(V, DP, D, B, B_tc):
    info = plsc.get_sparse_core_info()
    NC, NS = info.num_cores, info.num_subcores
    NW = NC * NS
    b_per_w = B // NW
    w_split = B_tc // b_per_w
    assert B % (8 * NW) == 0 and B_tc % b_per_w == 0 and DP % info.num_lanes == 0
    mesh = plsc.VectorSubcoreMesh(core_axis_name="c", subcore_axis_name="s")

    @functools.partial(
        pl.kernel,
        mesh=mesh,
        out_type=jax.ShapeDtypeStruct((B, DP), jnp.float32),
        scratch_types=[
            pltpu.VMEM((b_per_w,), jnp.int32),
            pltpu.VMEM((b_per_w, DP), jnp.float32),
            pltpu.SemaphoreType.DMA,
        ],
    )
    def gather(table_hbm, idx_tc_hbm, idx_sc_hbm, out_hbm, idx_v, rows_v, sem):
        wid = lax.axis_index("s") * NC + lax.axis_index("c")
        base = wid * b_per_w

        @pl.when(wid < w_split)
        def _():
            pltpu.sync_copy(idx_tc_hbm.at[pl.ds(base, b_per_w)], idx_v)

        @pl.when(wid >= w_split)
        def _():
            pltpu.sync_copy(
                idx_sc_hbm.at[pl.ds(base - B_tc, b_per_w)], idx_v
            )

        pltpu.async_copy(table_hbm.at[idx_v], rows_v, sem).wait()
        pltpu.sync_copy(rows_v, out_hbm.at[pl.ds(base, b_per_w)])

    return gather


@jax.jit
def kernel(x, dictionary):
    B, N, V = x.shape
    D = dictionary.shape[1]
    R = B * N
    xf = x.reshape(R, V)

    S = _SC_ROWS
    R_tc = R - S
    tokens_tc = pl.pallas_call(
        _argmax_body,
        grid=(R_tc // _TC_BLK,),
        in_specs=[pl.BlockSpec((_TC_BLK, V), lambda i: (i, 0))],
        out_specs=pl.BlockSpec((_TC_BLK,), lambda i: (i,)),
        out_shape=jax.ShapeDtypeStruct((R_tc,), jnp.int32),
    )(xf)
    tokens_sc = _make_sc_argmax(S, V, R_tc)(xf)

    DP = 128
    table = jnp.pad(dictionary, ((0, 0), (0, DP - D)))
    out = _make_sc_gather(V, DP, D, R, R_tc)(table, tokens_tc, tokens_sc)
    return out[:, :D].reshape(B, N, D)
